# Initial kernel scaffold; baseline (speedup 1.0000x reference)
import jax
import jax.numpy as jnp
from jax.experimental import pallas as pl


def kernel(x, norm, gamma, beta, edge_index):
    raise NotImplementedError("write your pallas kernel here")


# speedup vs baseline: 5.1499x; 5.1499x over previous
import jax
import jax.numpy as jnp
from jax import lax
from jax.experimental import pallas as pl
from jax.experimental.pallas import tpu as pltpu
from jax.experimental.pallas import tpu_sc as plsc

N = 10000
E = 160000
D = 256
H = 128
NS = 16
EPT = E // NS
CH = 80
NCH = EPT // CH
NST = 5
IB = NCH // NST
RPT = 632
RPT_L = N - (NS - 1) * RPT
HR = 80
L = 16
RB = 1000
NB = N // RB


def _scale_body(x_ref, norm_ref, h_ref):
    h = x_ref[...] * norm_ref[...]
    h_ref[0, :, :] = h[:, :H]
    h_ref[1, :, :] = h[:, H:]


def _scale(x, norm):
    return pl.pallas_call(
        _scale_body,
        grid=(NB,),
        in_specs=[
            pl.BlockSpec((RB, D), lambda i: (i, 0)),
            pl.BlockSpec((RB, 1), lambda i: (i, 0)),
        ],
        out_specs=pl.BlockSpec((2, RB, H), lambda i: (0, i, 0)),
        out_shape=jax.ShapeDtypeStruct((2, N, H), jnp.float32),
    )(x, norm)


def _sc_agg_body(h0, h1, edg, s0, s1, deg,
                 src_v, dst_v, rows_v, hist1, idx80, zbuf, acc_sh, sem):
    c = lax.axis_index("c")
    tid = lax.axis_index("s")
    r0 = pl.multiple_of(tid * RPT, 8)
    nz = NS - 1

    zv = jnp.zeros((L,), jnp.float32)
    ov = jnp.ones((L,), jnp.float32)
    iv = lax.iota(jnp.int32, L)
    for i in range(8):
        for k in range(H // L):
            zbuf[i, k * L:(k + 1) * L] = zv
    for k in range(HR // L):
        idx80[k * L:(k + 1) * L] = iv + (k * L)

    def zh(j, carry):
        hist1[pl.ds(j * L, L)] = zv
        return carry
    lax.fori_loop(0, (HR * H) // L, zh, 0)

    def z8(j, carry):
        rj = pl.multiple_of(r0 + j * 8, 8)
        pltpu.sync_copy(zbuf, acc_sh.at[pl.ds(rj, 8)])
        return carry

    @pl.when(tid < nz)
    def _():
        lax.fori_loop(0, RPT // 8, z8, 0)

    @pl.when(tid == nz)
    def _():
        lax.fori_loop(0, RPT_L // 8, z8, 0)

    plsc.subcore_barrier()

    def main_loop(h_half, count_deg):
        def stage(g, carry):
            pltpu.sync_copy(edg.at[0, tid, g], src_v)
            pltpu.sync_copy(edg.at[1, tid, g], dst_v)

            def body(j, carry2):
                pltpu.async_copy(h_half.at[src_v.at[j]], rows_v, sem).wait()
                pltpu.sync_copy(rows_v, acc_sh.at[dst_v.at[j]], add=True)
                if count_deg:
                    for k in range(CH // L):
                        vec = dst_v[j, k * L:(k + 1) * L]
                        plsc.addupdate_scatter(hist1, [vec], ov)
                return carry2
            lax.fori_loop(0, IB, body, 0)
            return carry
        lax.fori_loop(0, NST, stage, 0)

    @pl.when(c == 0)
    def _():
        main_loop(h0, True)

    @pl.when(c == 1)
    def _():
        main_loop(h1, False)

    plsc.subcore_barrier()

    def write_out(cnt):
        @pl.when(c == 0)
        def _():
            pltpu.sync_copy(acc_sh.at[pl.ds(r0, cnt)], s0.at[pl.ds(r0, cnt)])

        @pl.when(c == 1)
        def _():
            pltpu.sync_copy(acc_sh.at[pl.ds(r0, cnt)], s1.at[pl.ds(r0, cnt)])

    @pl.when(tid < nz)
    def _():
        write_out(RPT)

    @pl.when(tid == nz)
    def _():
        write_out(RPT_L)

    @pl.when(c == 0)
    def _():
        @pl.when(tid == 0)
        def _():
            def zd(j, carry):
                rj = pl.multiple_of(j * 8, 8)
                pltpu.sync_copy(zbuf, acc_sh.at[pl.ds(rj, 8)])
                return carry
            lax.fori_loop(0, HR // 8, zd, 0)

        plsc.subcore_barrier()

        def cp(j, carry):
            for k in range(H // L):
                rows_v[j, k * L:(k + 1) * L] = hist1[pl.ds(j * H + k * L, L)]
            return carry
        lax.fori_loop(0, HR, cp, 0)
        pltpu.sync_copy(rows_v.at[pl.ds(0, HR)], acc_sh.at[idx80], add=True)

        plsc.subcore_barrier()

        @pl.when(tid == 0)
        def _():
            pltpu.sync_copy(acc_sh.at[pl.ds(0, HR)], deg)


def _sc_agg(h0, h1, edg):
    mesh = plsc.VectorSubcoreMesh(core_axis_name="c", subcore_axis_name="s",
                                  num_cores=2, num_subcores=NS)
    f = pl.kernel(
        _sc_agg_body,
        out_type=(
            jax.ShapeDtypeStruct((N, H), jnp.float32),
            jax.ShapeDtypeStruct((N, H), jnp.float32),
            jax.ShapeDtypeStruct((HR, H), jnp.float32),
        ),
        mesh=mesh,
        compiler_params=pltpu.CompilerParams(needs_layout_passes=False),
        scratch_types=[
            pltpu.VMEM((IB, CH), jnp.int32),
            pltpu.VMEM((IB, CH), jnp.int32),
            pltpu.VMEM((CH, H), jnp.float32),
            pltpu.VMEM((HR * H,), jnp.float32),
            pltpu.VMEM((HR,), jnp.int32),
            pltpu.VMEM((8, H), jnp.float32),
            pltpu.VMEM_SHARED((N, H), jnp.float32),
            pltpu.SemaphoreType.DMA,
        ],
    )
    return f(h0, h1, edg)


def _post_body(s2x_ref, deg_ref, h2x_ref, norm_ref, b_ref, h2_ref, stats_ref):
    i = pl.program_id(0)

    @pl.when(i == 0)
    def _():
        stats_ref[...] = jnp.zeros((8, D), jnp.float32)

    dinv = 1.0 / jnp.maximum(deg_ref[...], 1.0)
    c0 = s2x_ref[0, :, :] * dinv
    c1 = s2x_ref[1, :, :] * dinv
    h0 = h2x_ref[0, :, :]
    h1 = h2x_ref[1, :, :]
    ssq = (jnp.sum(h0 * h0, axis=1, keepdims=True)
           + jnp.sum(h1 * h1, axis=1, keepdims=True)
           + jnp.sum(c0 * c0, axis=1, keepdims=True)
           + jnp.sum(c1 * c1, axis=1, keepdims=True))
    inv = 1.0 / jnp.maximum(jnp.sqrt(ssq), 1e-12)
    b_ref[:, 0 * H:1 * H] = h0 * inv
    b_ref[:, 1 * H:2 * H] = h1 * inv
    b_ref[:, 2 * H:3 * H] = c0 * inv
    b_ref[:, 3 * H:4 * H] = c1 * inv

    nrm = norm_ref[...]
    h2 = jnp.concatenate([c0 * nrm, c1 * nrm], axis=1)
    h2_ref[...] = h2
    stats_ref[0:1, :] += jnp.sum(h2, axis=0, keepdims=True)
    stats_ref[1:2, :] += jnp.sum(h2 * h2, axis=0, keepdims=True)


def _post(s2x, deg, h2x, norm):
    return pl.pallas_call(
        _post_body,
        grid=(NB,),
        in_specs=[
            pl.BlockSpec((2, RB, H), lambda i: (0, i, 0)),
            pl.BlockSpec((RB, 1), lambda i: (i, 0)),
            pl.BlockSpec((2, RB, H), lambda i: (0, i, 0)),
            pl.BlockSpec((RB, 1), lambda i: (i, 0)),
        ],
        out_specs=[
            pl.BlockSpec((RB, 2 * D), lambda i: (i, 0)),
            pl.BlockSpec((RB, D), lambda i: (i, 0)),
            pl.BlockSpec((8, D), lambda i: (0, 0)),
        ],
        out_shape=[
            jax.ShapeDtypeStruct((N, 2 * D), jnp.float32),
            jax.ShapeDtypeStruct((N, D), jnp.float32),
            jax.ShapeDtypeStruct((8, D), jnp.float32),
        ],
    )(s2x, deg, h2x, norm)


def _bn_body(h2_ref, stats_ref, gamma_ref, beta_ref, h3_ref):
    mean = stats_ref[0:1, :] / float(N)
    var = stats_ref[1:2, :] / float(N) - mean * mean
    scale = gamma_ref[...] * lax.rsqrt(var + 1e-5)
    h3_ref[...] = (h2_ref[...] - mean) * scale + beta_ref[...]


def _bn(h2, stats, gamma, beta):
    return pl.pallas_call(
        _bn_body,
        grid=(NB,),
        in_specs=[
            pl.BlockSpec((RB, D), lambda i: (i, 0)),
            pl.BlockSpec((8, D), lambda i: (0, 0)),
            pl.BlockSpec((1, D), lambda i: (0, 0)),
            pl.BlockSpec((1, D), lambda i: (0, 0)),
        ],
        out_specs=pl.BlockSpec((RB, D), lambda i: (i, 0)),
        out_shape=jax.ShapeDtypeStruct((N, D), jnp.float32),
    )(h2, stats, gamma, beta)


def kernel(x, norm, gamma, beta, edge_index):
    edg = edge_index.astype(jnp.int32).reshape(2, NS, NST, IB, CH)
    h2x = _scale(x, norm)
    s0, s1, degq = _sc_agg(h2x[0], h2x[1], edg)
    s2x = jnp.stack([s0, s1])
    deg = degq.reshape(HR * H)[:N].reshape(N, 1)
    b, h2, stats = _post(s2x, deg, h2x, norm)
    h3 = _bn(h2, stats, gamma.reshape(1, D), beta.reshape(1, D))
    return (h3, b)


# speedup vs baseline: 5.5529x; 1.0783x over previous
import jax
import jax.numpy as jnp
from jax import lax
from jax.experimental import pallas as pl
from jax.experimental.pallas import tpu as pltpu
from jax.experimental.pallas import tpu_sc as plsc

N = 10000
E = 160000
D = 256
H = 128
NS = 16
EPT = E // NS
CH = 80
NCH = EPT // CH
NST = 25
IB = NCH // NST
RPT = 632
RPT_L = N - (NS - 1) * RPT
HR = 80
L = 16
RB = 1000
NB = N // RB


def _scale_body(x_ref, norm_ref, h_ref):
    h = x_ref[...] * norm_ref[...]
    h_ref[0, :, :] = h[:, :H]
    h_ref[1, :, :] = h[:, H:]


def _scale(x, norm):
    return pl.pallas_call(
        _scale_body,
        grid=(NB,),
        in_specs=[
            pl.BlockSpec((RB, D), lambda i: (i, 0)),
            pl.BlockSpec((RB, 1), lambda i: (i, 0)),
        ],
        out_specs=pl.BlockSpec((2, RB, H), lambda i: (0, i, 0)),
        out_shape=jax.ShapeDtypeStruct((2, N, H), jnp.float32),
    )(x, norm)


def _sc_agg_body(h0, h1, edg, s0, s1, deg,
                 src_v, dst_v, rows_a, rows_b, hist1, idx80, acc_sh, sem):
    c = lax.axis_index("c")
    tid = lax.axis_index("s")
    r0 = pl.multiple_of(tid * RPT, 8)
    nz = NS - 1

    zv = jnp.zeros((L,), jnp.float32)
    ov = jnp.ones((L,), jnp.float32)
    iv = lax.iota(jnp.int32, L)
    for i in range(8):
        for k in range(H // L):
            rows_a[i, k * L:(k + 1) * L] = zv
    for k in range(HR // L):
        idx80[k * L:(k + 1) * L] = iv + (k * L)

    def zh(j, carry):
        hist1[pl.ds(j * L, L)] = zv
        return carry
    lax.fori_loop(0, (HR * H) // L, zh, 0)

    def z8(j, carry):
        rj = pl.multiple_of(r0 + j * 8, 8)
        pltpu.sync_copy(rows_a.at[pl.ds(0, 8)], acc_sh.at[pl.ds(rj, 8)])
        return carry

    @pl.when(tid < nz)
    def _():
        lax.fori_loop(0, RPT // 8, z8, 0)

    @pl.when(tid == nz)
    def _():
        lax.fori_loop(0, RPT_L // 8, z8, 0)

    plsc.subcore_barrier()

    def main_loop(h_half, count_deg):
        bufs = [rows_a, rows_b]

        def stage(g, carry):
            pltpu.sync_copy(edg.at[0, tid, g], src_v)
            pltpu.sync_copy(edg.at[1, tid, g], dst_v)

            pending = pltpu.async_copy(h_half.at[src_v.at[0]], bufs[0], sem)
            for j in range(IB):
                pending.wait()
                if j + 1 < IB:
                    pending = pltpu.async_copy(
                        h_half.at[src_v.at[j + 1]], bufs[(j + 1) % 2], sem)
                pltpu.sync_copy(bufs[j % 2], acc_sh.at[dst_v.at[j]], add=True)
                if count_deg:
                    for k in range(CH // L):
                        vec = dst_v[j, k * L:(k + 1) * L]
                        plsc.addupdate_scatter(hist1, [vec], ov)
            return carry
        lax.fori_loop(0, NST, stage, 0)

    @pl.when(c == 0)
    def _():
        main_loop(h0, True)

    @pl.when(c == 1)
    def _():
        main_loop(h1, False)

    plsc.subcore_barrier()

    def write_out(cnt):
        @pl.when(c == 0)
        def _():
            pltpu.sync_copy(acc_sh.at[pl.ds(r0, cnt)], s0.at[pl.ds(r0, cnt)])

        @pl.when(c == 1)
        def _():
            pltpu.sync_copy(acc_sh.at[pl.ds(r0, cnt)], s1.at[pl.ds(r0, cnt)])

    @pl.when(tid < nz)
    def _():
        write_out(RPT)

    @pl.when(tid == nz)
    def _():
        write_out(RPT_L)

    @pl.when(c == 0)
    def _():
        @pl.when(tid == 0)
        def _():
            for i in range(8):
                for k in range(H // L):
                    rows_b[i, k * L:(k + 1) * L] = zv
            def zd(j, carry):
                rj = pl.multiple_of(j * 8, 8)
                pltpu.sync_copy(rows_b.at[pl.ds(0, 8)], acc_sh.at[pl.ds(rj, 8)])
                return carry
            lax.fori_loop(0, HR // 8, zd, 0)

        plsc.subcore_barrier()

        def cp(j, carry):
            for k in range(H // L):
                rows_a[j, k * L:(k + 1) * L] = hist1[pl.ds(j * H + k * L, L)]
            return carry
        lax.fori_loop(0, HR, cp, 0)
        pltpu.sync_copy(rows_a.at[pl.ds(0, HR)], acc_sh.at[idx80], add=True)

        plsc.subcore_barrier()

        @pl.when(tid == 0)
        def _():
            pltpu.sync_copy(acc_sh.at[pl.ds(0, HR)], deg)


def _sc_agg(h0, h1, edg):
    mesh = plsc.VectorSubcoreMesh(core_axis_name="c", subcore_axis_name="s",
                                  num_cores=2, num_subcores=NS)
    f = pl.kernel(
        _sc_agg_body,
        out_type=(
            jax.ShapeDtypeStruct((N, H), jnp.float32),
            jax.ShapeDtypeStruct((N, H), jnp.float32),
            jax.ShapeDtypeStruct((HR, H), jnp.float32),
        ),
        mesh=mesh,
        compiler_params=pltpu.CompilerParams(needs_layout_passes=False),
        scratch_types=[
            pltpu.VMEM((IB, CH), jnp.int32),
            pltpu.VMEM((IB, CH), jnp.int32),
            pltpu.VMEM((CH, H), jnp.float32),
            pltpu.VMEM((CH, H), jnp.float32),
            pltpu.VMEM((HR * H,), jnp.float32),
            pltpu.VMEM((HR,), jnp.int32),
            pltpu.VMEM_SHARED((N, H), jnp.float32),
            pltpu.SemaphoreType.DMA,
        ],
    )
    return f(h0, h1, edg)


def _post_body(s2x_ref, deg_ref, h2x_ref, norm_ref, b_ref, h2_ref, stats_ref):
    i = pl.program_id(0)

    @pl.when(i == 0)
    def _():
        stats_ref[...] = jnp.zeros((8, D), jnp.float32)

    dinv = 1.0 / jnp.maximum(deg_ref[...], 1.0)
    c0 = s2x_ref[0, :, :] * dinv
    c1 = s2x_ref[1, :, :] * dinv
    h0 = h2x_ref[0, :, :]
    h1 = h2x_ref[1, :, :]
    ssq = (jnp.sum(h0 * h0, axis=1, keepdims=True)
           + jnp.sum(h1 * h1, axis=1, keepdims=True)
           + jnp.sum(c0 * c0, axis=1, keepdims=True)
           + jnp.sum(c1 * c1, axis=1, keepdims=True))
    inv = 1.0 / jnp.maximum(jnp.sqrt(ssq), 1e-12)
    b_ref[:, 0 * H:1 * H] = h0 * inv
    b_ref[:, 1 * H:2 * H] = h1 * inv
    b_ref[:, 2 * H:3 * H] = c0 * inv
    b_ref[:, 3 * H:4 * H] = c1 * inv

    nrm = norm_ref[...]
    h2 = jnp.concatenate([c0 * nrm, c1 * nrm], axis=1)
    h2_ref[...] = h2
    stats_ref[0:1, :] += jnp.sum(h2, axis=0, keepdims=True)
    stats_ref[1:2, :] += jnp.sum(h2 * h2, axis=0, keepdims=True)


def _post(s2x, deg, h2x, norm):
    return pl.pallas_call(
        _post_body,
        grid=(NB,),
        in_specs=[
            pl.BlockSpec((2, RB, H), lambda i: (0, i, 0)),
            pl.BlockSpec((RB, 1), lambda i: (i, 0)),
            pl.BlockSpec((2, RB, H), lambda i: (0, i, 0)),
            pl.BlockSpec((RB, 1), lambda i: (i, 0)),
        ],
        out_specs=[
            pl.BlockSpec((RB, 2 * D), lambda i: (i, 0)),
            pl.BlockSpec((RB, D), lambda i: (i, 0)),
            pl.BlockSpec((8, D), lambda i: (0, 0)),
        ],
        out_shape=[
            jax.ShapeDtypeStruct((N, 2 * D), jnp.float32),
            jax.ShapeDtypeStruct((N, D), jnp.float32),
            jax.ShapeDtypeStruct((8, D), jnp.float32),
        ],
    )(s2x, deg, h2x, norm)


def _bn_body(h2_ref, stats_ref, gamma_ref, beta_ref, h3_ref):
    mean = stats_ref[0:1, :] / float(N)
    var = stats_ref[1:2, :] / float(N) - mean * mean
    scale = gamma_ref[...] * lax.rsqrt(var + 1e-5)
    h3_ref[...] = (h2_ref[...] - mean) * scale + beta_ref[...]


def _bn(h2, stats, gamma, beta):
    return pl.pallas_call(
        _bn_body,
        grid=(NB,),
        in_specs=[
            pl.BlockSpec((RB, D), lambda i: (i, 0)),
            pl.BlockSpec((8, D), lambda i: (0, 0)),
            pl.BlockSpec((1, D), lambda i: (0, 0)),
            pl.BlockSpec((1, D), lambda i: (0, 0)),
        ],
        out_specs=pl.BlockSpec((RB, D), lambda i: (i, 0)),
        out_shape=jax.ShapeDtypeStruct((N, D), jnp.float32),
    )(h2, stats, gamma, beta)


def kernel(x, norm, gamma, beta, edge_index):
    edg = edge_index.astype(jnp.int32).reshape(2, NS, NST, IB, CH)
    h2x = _scale(x, norm)
    s0, s1, degq = _sc_agg(h2x[0], h2x[1], edg)
    s2x = jnp.stack([s0, s1])
    deg = degq.reshape(HR * H)[:N].reshape(N, 1)
    b, h2, stats = _post(s2x, deg, h2x, norm)
    h3 = _bn(h2, stats, gamma.reshape(1, D), beta.reshape(1, D))
    return (h3, b)


# speedup vs baseline: 5.6762x; 1.0222x over previous
import jax
import jax.numpy as jnp
from jax import lax
from jax.experimental import pallas as pl
from jax.experimental.pallas import tpu as pltpu
from jax.experimental.pallas import tpu_sc as plsc

N = 10000
E = 160000
D = 256
H = 128
NS = 16
EPT = E // NS
CH = 80
NCH = EPT // CH
NST = 25
IB = NCH // NST
RPT = 632
RPT_L = N - (NS - 1) * RPT
HR = 80
L = 16
RB = 1000
NB = N // RB


def _scale_body(x_ref, norm_ref, h_ref):
    h = x_ref[...] * norm_ref[...]
    h_ref[0, :, :] = h[:, :H]
    h_ref[1, :, :] = h[:, H:]


def _scale(x, norm):
    return pl.pallas_call(
        _scale_body,
        grid=(NB,),
        in_specs=[
            pl.BlockSpec((RB, D), lambda i: (i, 0)),
            pl.BlockSpec((RB, 1), lambda i: (i, 0)),
        ],
        out_specs=pl.BlockSpec((2, RB, H), lambda i: (0, i, 0)),
        out_shape=jax.ShapeDtypeStruct((2, N, H), jnp.float32),
    )(x, norm)


def _sc_agg_body(h0, h1, edg, s0, s1, deg,
                 src_v, dst_v, rows_a, rows_b, hist1, idx80, acc_sh, sem):
    c = lax.axis_index("c")
    tid = lax.axis_index("s")
    r0 = pl.multiple_of(tid * RPT, 8)
    nz = NS - 1

    zv = jnp.zeros((L,), jnp.float32)
    ov = jnp.ones((L,), jnp.float32)
    iv = lax.iota(jnp.int32, L)
    for i in range(8):
        for k in range(H // L):
            rows_a[i, k * L:(k + 1) * L] = zv
    for k in range(HR // L):
        idx80[k * L:(k + 1) * L] = iv + (k * L)

    def zh(j, carry):
        hist1[pl.ds(j * L, L)] = zv
        return carry
    lax.fori_loop(0, (HR * H) // L, zh, 0)

    def z8(j, carry):
        rj = pl.multiple_of(r0 + j * 8, 8)
        pltpu.sync_copy(rows_a.at[pl.ds(0, 8)], acc_sh.at[pl.ds(rj, 8)])
        return carry

    @pl.when(tid < nz)
    def _():
        lax.fori_loop(0, RPT // 8, z8, 0)

    @pl.when(tid == nz)
    def _():
        lax.fori_loop(0, RPT_L // 8, z8, 0)

    plsc.subcore_barrier()

    def main_loop(h_half, count_deg):
        bufs = [rows_a, rows_b]

        def stage(g, carry):
            pltpu.sync_copy(edg.at[0, tid, g], src_v)
            pltpu.sync_copy(edg.at[1, tid, g], dst_v)

            pending = pltpu.async_copy(h_half.at[src_v.at[0]], bufs[0], sem)
            for j in range(IB):
                pending.wait()
                if j + 1 < IB:
                    pending = pltpu.async_copy(
                        h_half.at[src_v.at[j + 1]], bufs[(j + 1) % 2], sem)
                pltpu.sync_copy(bufs[j % 2], acc_sh.at[dst_v.at[j]], add=True)
                if count_deg:
                    for k in range(CH // L):
                        vec = dst_v[j, k * L:(k + 1) * L]
                        plsc.addupdate_scatter(hist1, [vec], ov)
            return carry
        lax.fori_loop(0, NST, stage, 0)

    @pl.when(c == 0)
    def _():
        main_loop(h0, True)

    @pl.when(c == 1)
    def _():
        main_loop(h1, False)

    plsc.subcore_barrier()

    def write_out(cnt):
        @pl.when(c == 0)
        def _():
            pltpu.sync_copy(acc_sh.at[pl.ds(r0, cnt)], s0.at[pl.ds(r0, cnt)])

        @pl.when(c == 1)
        def _():
            pltpu.sync_copy(acc_sh.at[pl.ds(r0, cnt)], s1.at[pl.ds(r0, cnt)])

    @pl.when(tid < nz)
    def _():
        write_out(RPT)

    @pl.when(tid == nz)
    def _():
        write_out(RPT_L)

    @pl.when(c == 0)
    def _():
        @pl.when(tid == 0)
        def _():
            for i in range(8):
                for k in range(H // L):
                    rows_b[i, k * L:(k + 1) * L] = zv
            def zd(j, carry):
                rj = pl.multiple_of(j * 8, 8)
                pltpu.sync_copy(rows_b.at[pl.ds(0, 8)], acc_sh.at[pl.ds(rj, 8)])
                return carry
            lax.fori_loop(0, HR // 8, zd, 0)

        plsc.subcore_barrier()

        def cp(j, carry):
            for k in range(H // L):
                rows_a[j, k * L:(k + 1) * L] = hist1[pl.ds(j * H + k * L, L)]
            return carry
        lax.fori_loop(0, HR, cp, 0)
        pltpu.sync_copy(rows_a.at[pl.ds(0, HR)], acc_sh.at[idx80], add=True)

        plsc.subcore_barrier()

        @pl.when(tid == 0)
        def _():
            pltpu.sync_copy(acc_sh.at[pl.ds(0, HR)], deg)


def _sc_agg(h0, h1, edg):
    mesh = plsc.VectorSubcoreMesh(core_axis_name="c", subcore_axis_name="s",
                                  num_cores=2, num_subcores=NS)
    f = pl.kernel(
        _sc_agg_body,
        out_type=(
            jax.ShapeDtypeStruct((N, H), jnp.float32),
            jax.ShapeDtypeStruct((N, H), jnp.float32),
            jax.ShapeDtypeStruct((HR, H), jnp.float32),
        ),
        mesh=mesh,
        compiler_params=pltpu.CompilerParams(needs_layout_passes=False),
        scratch_types=[
            pltpu.VMEM((IB, CH), jnp.int32),
            pltpu.VMEM((IB, CH), jnp.int32),
            pltpu.VMEM((CH, H), jnp.float32),
            pltpu.VMEM((CH, H), jnp.float32),
            pltpu.VMEM((HR * H,), jnp.float32),
            pltpu.VMEM((HR,), jnp.int32),
            pltpu.VMEM_SHARED((N, H), jnp.float32),
            pltpu.SemaphoreType.DMA,
        ],
    )
    return f(h0, h1, edg)


def _post_body(s0_ref, s1_ref, deg_ref, h2x_ref, norm_ref, b_ref, h2_ref, stats_ref):
    i = pl.program_id(0)

    @pl.when(i == 0)
    def _():
        stats_ref[...] = jnp.zeros((8, D), jnp.float32)

    dinv = 1.0 / jnp.maximum(deg_ref[...], 1.0)
    c0 = s0_ref[...] * dinv
    c1 = s1_ref[...] * dinv
    h0 = h2x_ref[0, :, :]
    h1 = h2x_ref[1, :, :]
    ssq = (jnp.sum(h0 * h0, axis=1, keepdims=True)
           + jnp.sum(h1 * h1, axis=1, keepdims=True)
           + jnp.sum(c0 * c0, axis=1, keepdims=True)
           + jnp.sum(c1 * c1, axis=1, keepdims=True))
    inv = 1.0 / jnp.maximum(jnp.sqrt(ssq), 1e-12)
    b_ref[:, 0 * H:1 * H] = h0 * inv
    b_ref[:, 1 * H:2 * H] = h1 * inv
    b_ref[:, 2 * H:3 * H] = c0 * inv
    b_ref[:, 3 * H:4 * H] = c1 * inv

    nrm = norm_ref[...]
    h2 = jnp.concatenate([c0 * nrm, c1 * nrm], axis=1)
    h2_ref[...] = h2
    stats_ref[0:1, :] += jnp.sum(h2, axis=0, keepdims=True)
    stats_ref[1:2, :] += jnp.sum(h2 * h2, axis=0, keepdims=True)


def _post(s0, s1, deg, h2x, norm):
    return pl.pallas_call(
        _post_body,
        grid=(NB,),
        in_specs=[
            pl.BlockSpec((RB, H), lambda i: (i, 0)),
            pl.BlockSpec((RB, H), lambda i: (i, 0)),
            pl.BlockSpec((RB, 1), lambda i: (i, 0)),
            pl.BlockSpec((2, RB, H), lambda i: (0, i, 0)),
            pl.BlockSpec((RB, 1), lambda i: (i, 0)),
        ],
        out_specs=[
            pl.BlockSpec((RB, 2 * D), lambda i: (i, 0)),
            pl.BlockSpec((RB, D), lambda i: (i, 0)),
            pl.BlockSpec((8, D), lambda i: (0, 0)),
        ],
        out_shape=[
            jax.ShapeDtypeStruct((N, 2 * D), jnp.float32),
            jax.ShapeDtypeStruct((N, D), jnp.float32),
            jax.ShapeDtypeStruct((8, D), jnp.float32),
        ],
    )(s0, s1, deg, h2x, norm)


def _bn_body(h2_ref, stats_ref, gamma_ref, beta_ref, h3_ref):
    mean = stats_ref[0:1, :] / float(N)
    var = stats_ref[1:2, :] / float(N) - mean * mean
    scale = gamma_ref[...] * lax.rsqrt(var + 1e-5)
    h3_ref[...] = (h2_ref[...] - mean) * scale + beta_ref[...]


def _bn(h2, stats, gamma, beta):
    return pl.pallas_call(
        _bn_body,
        grid=(NB,),
        in_specs=[
            pl.BlockSpec((RB, D), lambda i: (i, 0)),
            pl.BlockSpec((8, D), lambda i: (0, 0)),
            pl.BlockSpec((1, D), lambda i: (0, 0)),
            pl.BlockSpec((1, D), lambda i: (0, 0)),
        ],
        out_specs=pl.BlockSpec((RB, D), lambda i: (i, 0)),
        out_shape=jax.ShapeDtypeStruct((N, D), jnp.float32),
    )(h2, stats, gamma, beta)


def kernel(x, norm, gamma, beta, edge_index):
    edg = edge_index.astype(jnp.int32).reshape(2, NS, NST, IB, CH)
    h2x = _scale(x, norm)
    s0, s1, degq = _sc_agg(h2x[0], h2x[1], edg)
    deg = degq.reshape(HR * H)[:N].reshape(N, 1)
    b, h2, stats = _post(s0, s1, deg, h2x, norm)
    h3 = _bn(h2, stats, gamma.reshape(1, D), beta.reshape(1, D))
    return (h3, b)


# speedup vs baseline: 6.2291x; 1.0974x over previous
import jax
import jax.numpy as jnp
from jax import lax
from jax.experimental import pallas as pl
from jax.experimental.pallas import tpu as pltpu
from jax.experimental.pallas import tpu_sc as plsc

N = 10000
E = 160000
D = 256
H = 128
NS = 16
EPT = E // NS
CH = 80
NCH = EPT // CH
NST = 25
IB = NCH // NST
RPT = 632
RPT_L = N - (NS - 1) * RPT
HR = 80
L = 16
RB = 1000
NB = N // RB


def _scale_body(x_ref, norm_ref, h_ref):
    h = x_ref[...] * norm_ref[...]
    h_ref[0, :, :] = h[:, :H]
    h_ref[1, :, :] = h[:, H:]


def _scale(x, norm):
    return pl.pallas_call(
        _scale_body,
        grid=(NB,),
        in_specs=[
            pl.BlockSpec((RB, D), lambda i: (i, 0)),
            pl.BlockSpec((RB, 1), lambda i: (i, 0)),
        ],
        out_specs=pl.BlockSpec((2, RB, H), lambda i: (0, i, 0)),
        out_shape=jax.ShapeDtypeStruct((2, N, H), jnp.float32),
    )(x, norm)


def _sc_agg_body(h0, h1, edg, s0, s1, deg,
                 src_a, src_b, dst_a, dst_b, rows_a, rows_b, hist1, idx80,
                 acc_sh, sem, semi):
    c = lax.axis_index("c")
    tid = lax.axis_index("s")
    r0 = pl.multiple_of(tid * RPT, 8)
    nz = NS - 1

    zv = jnp.zeros((L,), jnp.float32)
    ov = jnp.ones((L,), jnp.float32)
    iv = lax.iota(jnp.int32, L)
    for i in range(8):
        for k in range(H // L):
            rows_a[i, k * L:(k + 1) * L] = zv
    for k in range(HR // L):
        idx80[k * L:(k + 1) * L] = iv + (k * L)

    def zh(j, carry):
        hist1[pl.ds(j * L, L)] = zv
        return carry
    lax.fori_loop(0, (HR * H) // L, zh, 0)

    def z8(j, carry):
        rj = pl.multiple_of(r0 + j * 8, 8)
        pltpu.sync_copy(rows_a.at[pl.ds(0, 8)], acc_sh.at[pl.ds(rj, 8)])
        return carry

    @pl.when(tid < nz)
    def _():
        lax.fori_loop(0, RPT // 8, z8, 0)

    @pl.when(tid == nz)
    def _():
        lax.fori_loop(0, RPT_L // 8, z8, 0)

    pltpu.async_copy(edg.at[0, tid, 0], src_a, semi)
    pltpu.async_copy(edg.at[1, tid, 0], dst_a, semi)

    plsc.subcore_barrier()

    def main_loop(h_half, count_deg):
        bufs = [rows_a, rows_b]

        def process_stage(src_v, dst_v):
            pending = pltpu.async_copy(h_half.at[src_v.at[0]], bufs[0], sem)
            for j in range(IB):
                pending.wait()
                if j + 1 < IB:
                    pending = pltpu.async_copy(
                        h_half.at[src_v.at[j + 1]], bufs[(j + 1) % 2], sem)
                pltpu.sync_copy(bufs[j % 2], acc_sh.at[dst_v.at[j]], add=True)
                if count_deg:
                    for k in range(CH // L):
                        vec = dst_v[j, k * L:(k + 1) * L]
                        plsc.addupdate_scatter(hist1, [vec], ov)

        def wait_idx(g, sv, dv):
            pltpu.make_async_copy(edg.at[0, tid, g], sv, semi).wait()
            pltpu.make_async_copy(edg.at[1, tid, g], dv, semi).wait()

        def start_idx(g, sv, dv):
            pltpu.async_copy(edg.at[0, tid, g], sv, semi)
            pltpu.async_copy(edg.at[1, tid, g], dv, semi)

        def stage_pair(t, carry):
            g = 2 * t
            wait_idx(g, src_a, dst_a)
            start_idx(g + 1, src_b, dst_b)
            process_stage(src_a, dst_a)
            wait_idx(g + 1, src_b, dst_b)
            start_idx(g + 2, src_a, dst_a)
            process_stage(src_b, dst_b)
            return carry
        lax.fori_loop(0, (NST - 1) // 2, stage_pair, 0)
        wait_idx(NST - 1, src_a, dst_a)
        process_stage(src_a, dst_a)

    @pl.when(c == 0)
    def _():
        main_loop(h0, True)

    @pl.when(c == 1)
    def _():
        main_loop(h1, False)

    plsc.subcore_barrier()

    def write_out(cnt):
        @pl.when(c == 0)
        def _():
            pltpu.sync_copy(acc_sh.at[pl.ds(r0, cnt)], s0.at[pl.ds(r0, cnt)])

        @pl.when(c == 1)
        def _():
            pltpu.sync_copy(acc_sh.at[pl.ds(r0, cnt)], s1.at[pl.ds(r0, cnt)])

    @pl.when(tid < nz)
    def _():
        write_out(RPT)

    @pl.when(tid == nz)
    def _():
        write_out(RPT_L)

    @pl.when(c == 0)
    def _():
        @pl.when(tid == 0)
        def _():
            for i in range(8):
                for k in range(H // L):
                    rows_b[i, k * L:(k + 1) * L] = zv
            def zd(j, carry):
                rj = pl.multiple_of(j * 8, 8)
                pltpu.sync_copy(rows_b.at[pl.ds(0, 8)], acc_sh.at[pl.ds(rj, 8)])
                return carry
            lax.fori_loop(0, HR // 8, zd, 0)

        plsc.subcore_barrier()

        def cp(j, carry):
            for k in range(H // L):
                rows_a[j, k * L:(k + 1) * L] = hist1[pl.ds(j * H + k * L, L)]
            return carry
        lax.fori_loop(0, HR, cp, 0)
        pltpu.sync_copy(rows_a.at[pl.ds(0, HR)], acc_sh.at[idx80], add=True)

        plsc.subcore_barrier()

        @pl.when(tid == 0)
        def _():
            pltpu.sync_copy(acc_sh.at[pl.ds(0, HR)], deg)


def _sc_agg(h0, h1, edg):
    mesh = plsc.VectorSubcoreMesh(core_axis_name="c", subcore_axis_name="s",
                                  num_cores=2, num_subcores=NS)
    f = pl.kernel(
        _sc_agg_body,
        out_type=(
            jax.ShapeDtypeStruct((N, H), jnp.float32),
            jax.ShapeDtypeStruct((N, H), jnp.float32),
            jax.ShapeDtypeStruct((HR, H), jnp.float32),
        ),
        mesh=mesh,
        compiler_params=pltpu.CompilerParams(needs_layout_passes=False),
        scratch_types=[
            pltpu.VMEM((IB, CH), jnp.int32),
            pltpu.VMEM((IB, CH), jnp.int32),
            pltpu.VMEM((IB, CH), jnp.int32),
            pltpu.VMEM((IB, CH), jnp.int32),
            pltpu.VMEM((CH, H), jnp.float32),
            pltpu.VMEM((CH, H), jnp.float32),
            pltpu.VMEM((HR * H,), jnp.float32),
            pltpu.VMEM((HR,), jnp.int32),
            pltpu.VMEM_SHARED((N, H), jnp.float32),
            pltpu.SemaphoreType.DMA,
            pltpu.SemaphoreType.DMA,
        ],
    )
    return f(h0, h1, edg)


def _post_body(s0_ref, s1_ref, deg_ref, h2x_ref, norm_ref, b_ref, h2_ref, stats_ref):
    i = pl.program_id(0)

    @pl.when(i == 0)
    def _():
        stats_ref[...] = jnp.zeros((8, D), jnp.float32)

    dinv = 1.0 / jnp.maximum(deg_ref[...], 1.0)
    c0 = s0_ref[...] * dinv
    c1 = s1_ref[...] * dinv
    h0 = h2x_ref[0, :, :]
    h1 = h2x_ref[1, :, :]
    ssq = (jnp.sum(h0 * h0, axis=1, keepdims=True)
           + jnp.sum(h1 * h1, axis=1, keepdims=True)
           + jnp.sum(c0 * c0, axis=1, keepdims=True)
           + jnp.sum(c1 * c1, axis=1, keepdims=True))
    inv = 1.0 / jnp.maximum(jnp.sqrt(ssq), 1e-12)
    b_ref[:, 0 * H:1 * H] = h0 * inv
    b_ref[:, 1 * H:2 * H] = h1 * inv
    b_ref[:, 2 * H:3 * H] = c0 * inv
    b_ref[:, 3 * H:4 * H] = c1 * inv

    nrm = norm_ref[...]
    h2 = jnp.concatenate([c0 * nrm, c1 * nrm], axis=1)
    h2_ref[...] = h2
    stats_ref[0:1, :] += jnp.sum(h2, axis=0, keepdims=True)
    stats_ref[1:2, :] += jnp.sum(h2 * h2, axis=0, keepdims=True)


def _post(s0, s1, deg, h2x, norm):
    return pl.pallas_call(
        _post_body,
        grid=(NB,),
        in_specs=[
            pl.BlockSpec((RB, H), lambda i: (i, 0)),
            pl.BlockSpec((RB, H), lambda i: (i, 0)),
            pl.BlockSpec((RB, 1), lambda i: (i, 0)),
            pl.BlockSpec((2, RB, H), lambda i: (0, i, 0)),
            pl.BlockSpec((RB, 1), lambda i: (i, 0)),
        ],
        out_specs=[
            pl.BlockSpec((RB, 2 * D), lambda i: (i, 0)),
            pl.BlockSpec((RB, D), lambda i: (i, 0)),
            pl.BlockSpec((8, D), lambda i: (0, 0)),
        ],
        out_shape=[
            jax.ShapeDtypeStruct((N, 2 * D), jnp.float32),
            jax.ShapeDtypeStruct((N, D), jnp.float32),
            jax.ShapeDtypeStruct((8, D), jnp.float32),
        ],
    )(s0, s1, deg, h2x, norm)


def _bn_body(h2_ref, stats_ref, gamma_ref, beta_ref, h3_ref):
    mean = stats_ref[0:1, :] / float(N)
    var = stats_ref[1:2, :] / float(N) - mean * mean
    scale = gamma_ref[...] * lax.rsqrt(var + 1e-5)
    h3_ref[...] = (h2_ref[...] - mean) * scale + beta_ref[...]


def _bn(h2, stats, gamma, beta):
    return pl.pallas_call(
        _bn_body,
        grid=(NB,),
        in_specs=[
            pl.BlockSpec((RB, D), lambda i: (i, 0)),
            pl.BlockSpec((8, D), lambda i: (0, 0)),
            pl.BlockSpec((1, D), lambda i: (0, 0)),
            pl.BlockSpec((1, D), lambda i: (0, 0)),
        ],
        out_specs=pl.BlockSpec((RB, D), lambda i: (i, 0)),
        out_shape=jax.ShapeDtypeStruct((N, D), jnp.float32),
    )(h2, stats, gamma, beta)


def kernel(x, norm, gamma, beta, edge_index):
    edg = edge_index.astype(jnp.int32).reshape(2, NS, NST, IB, CH)
    h2x = _scale(x, norm)
    s0, s1, degq = _sc_agg(h2x[0], h2x[1], edg)
    deg = degq.reshape(HR * H)[:N].reshape(N, 1)
    b, h2, stats = _post(s0, s1, deg, h2x, norm)
    h3 = _bn(h2, stats, gamma.reshape(1, D), beta.reshape(1, D))
    return (h3, b)


# speedup vs baseline: 6.5176x; 1.0463x over previous
import jax
import jax.numpy as jnp
from jax import lax
from jax.experimental import pallas as pl
from jax.experimental.pallas import tpu as pltpu
from jax.experimental.pallas import tpu_sc as plsc

N = 10000
E = 160000
D = 256
H = 128
NS = 16
EPT = E // NS
CH = 80
NCH = EPT // CH
NST = 25
IB = NCH // NST
RPT = 632
RPT_L = N - (NS - 1) * RPT
HR = 80
L = 16
RB = 1000
NB = N // RB


def _scale_body(x_ref, norm_ref, h_ref):
    h = x_ref[...] * norm_ref[...]
    h_ref[0, :, :] = h[:, :H]
    h_ref[1, :, :] = h[:, H:]


def _scale(x, norm):
    return pl.pallas_call(
        _scale_body,
        grid=(NB,),
        in_specs=[
            pl.BlockSpec((RB, D), lambda i: (i, 0)),
            pl.BlockSpec((RB, 1), lambda i: (i, 0)),
        ],
        out_specs=pl.BlockSpec((2, RB, H), lambda i: (0, i, 0)),
        out_shape=jax.ShapeDtypeStruct((2, N, H), jnp.float32),
    )(x, norm)


def _sc_agg_body(h0, h1, edg, s0, s1, deg,
                 src_a, src_b, dst_a, dst_b, rows_a, rows_b, hist1, idx80,
                 acc_sh, sem, semi):
    c = lax.axis_index("c")
    tid = lax.axis_index("s")
    r0 = pl.multiple_of(tid * RPT, 8)
    nz = NS - 1

    zv = jnp.zeros((L,), jnp.float32)
    ov = jnp.ones((L,), jnp.float32)
    iv = lax.iota(jnp.int32, L)
    for i in range(8):
        for k in range(H // L):
            rows_a[i, k * L:(k + 1) * L] = zv
    for k in range(HR // L):
        idx80[k * L:(k + 1) * L] = iv + (k * L)

    def zh(j, carry):
        hist1[pl.ds(j * L, L)] = zv
        return carry
    lax.fori_loop(0, (HR * H) // L, zh, 0)

    def z8(j, carry):
        rj = pl.multiple_of(r0 + j * 8, 8)
        pltpu.sync_copy(rows_a.at[pl.ds(0, 8)], acc_sh.at[pl.ds(rj, 8)])
        return carry

    @pl.when(tid < nz)
    def _():
        lax.fori_loop(0, RPT // 8, z8, 0)

    @pl.when(tid == nz)
    def _():
        lax.fori_loop(0, RPT_L // 8, z8, 0)

    pltpu.async_copy(edg.at[0, tid, 0], src_a, semi)
    pltpu.async_copy(edg.at[1, tid, 0], dst_a, semi)

    plsc.subcore_barrier()

    def main_loop(h_half, count_deg):
        bufs = [rows_a, rows_b]

        def process_stage(src_v, dst_v, p, next_src, wait_fn):
            for j in range(IB):
                b = (p + j) % 2
                pltpu.make_async_copy(
                    h_half.at[src_v.at[j]], bufs[b], sem).wait()
                nxt = (p + j + 1) % 2
                if j + 1 < IB:
                    pltpu.async_copy(
                        h_half.at[src_v.at[j + 1]], bufs[nxt], sem)
                else:
                    if wait_fn is not None:
                        wait_fn()
                    if next_src is not None:
                        pltpu.async_copy(
                            h_half.at[next_src.at[0]], bufs[nxt], sem)
                pltpu.sync_copy(bufs[b], acc_sh.at[dst_v.at[j]], add=True)
                if count_deg:
                    for k in range(CH // L):
                        vec = dst_v[j, k * L:(k + 1) * L]
                        plsc.addupdate_scatter(hist1, [vec], ov)

        def wait_idx(g, sv, dv):
            pltpu.make_async_copy(edg.at[0, tid, g], sv, semi).wait()
            pltpu.make_async_copy(edg.at[1, tid, g], dv, semi).wait()

        def start_idx(g, sv, dv):
            pltpu.async_copy(edg.at[0, tid, g], sv, semi)
            pltpu.async_copy(edg.at[1, tid, g], dv, semi)

        wait_idx(0, src_a, dst_a)
        pltpu.async_copy(h_half.at[src_a.at[0]], bufs[0], sem)
        start_idx(1, src_b, dst_b)

        def stage_pair(t, carry):
            g = 2 * t
            process_stage(src_a, dst_a, 0, src_b,
                          lambda: wait_idx(g + 1, src_b, dst_b))
            start_idx(g + 2, src_a, dst_a)
            process_stage(src_b, dst_b, 1, src_a,
                          lambda: wait_idx(g + 2, src_a, dst_a))

            @pl.when(g + 3 < NST)
            def _():
                start_idx(g + 3, src_b, dst_b)
            return carry
        lax.fori_loop(0, (NST - 1) // 2, stage_pair, 0)
        process_stage(src_a, dst_a, 0, None, None)

    @pl.when(c == 0)
    def _():
        main_loop(h0, True)

    @pl.when(c == 1)
    def _():
        main_loop(h1, False)

    plsc.subcore_barrier()

    def write_out(cnt):
        @pl.when(c == 0)
        def _():
            pltpu.sync_copy(acc_sh.at[pl.ds(r0, cnt)], s0.at[pl.ds(r0, cnt)])

        @pl.when(c == 1)
        def _():
            pltpu.sync_copy(acc_sh.at[pl.ds(r0, cnt)], s1.at[pl.ds(r0, cnt)])

    @pl.when(tid < nz)
    def _():
        write_out(RPT)

    @pl.when(tid == nz)
    def _():
        write_out(RPT_L)

    @pl.when(c == 0)
    def _():
        @pl.when(tid == 0)
        def _():
            for i in range(8):
                for k in range(H // L):
                    rows_b[i, k * L:(k + 1) * L] = zv
            def zd(j, carry):
                rj = pl.multiple_of(j * 8, 8)
                pltpu.sync_copy(rows_b.at[pl.ds(0, 8)], acc_sh.at[pl.ds(rj, 8)])
                return carry
            lax.fori_loop(0, HR // 8, zd, 0)

        plsc.subcore_barrier()

        def cp(j, carry):
            for k in range(H // L):
                rows_a[j, k * L:(k + 1) * L] = hist1[pl.ds(j * H + k * L, L)]
            return carry
        lax.fori_loop(0, HR, cp, 0)
        pltpu.sync_copy(rows_a.at[pl.ds(0, HR)], acc_sh.at[idx80], add=True)

        plsc.subcore_barrier()

        @pl.when(tid == 0)
        def _():
            pltpu.sync_copy(acc_sh.at[pl.ds(0, HR)], deg)


def _sc_agg(h0, h1, edg):
    mesh = plsc.VectorSubcoreMesh(core_axis_name="c", subcore_axis_name="s",
                                  num_cores=2, num_subcores=NS)
    f = pl.kernel(
        _sc_agg_body,
        out_type=(
            jax.ShapeDtypeStruct((N, H), jnp.float32),
            jax.ShapeDtypeStruct((N, H), jnp.float32),
            jax.ShapeDtypeStruct((HR, H), jnp.float32),
        ),
        mesh=mesh,
        compiler_params=pltpu.CompilerParams(needs_layout_passes=False),
        scratch_types=[
            pltpu.VMEM((IB, CH), jnp.int32),
            pltpu.VMEM((IB, CH), jnp.int32),
            pltpu.VMEM((IB, CH), jnp.int32),
            pltpu.VMEM((IB, CH), jnp.int32),
            pltpu.VMEM((CH, H), jnp.float32),
            pltpu.VMEM((CH, H), jnp.float32),
            pltpu.VMEM((HR * H,), jnp.float32),
            pltpu.VMEM((HR,), jnp.int32),
            pltpu.VMEM_SHARED((N, H), jnp.float32),
            pltpu.SemaphoreType.DMA,
            pltpu.SemaphoreType.DMA,
        ],
    )
    return f(h0, h1, edg)


def _post_body(s0_ref, s1_ref, deg_ref, h2x_ref, norm_ref, b_ref, h2_ref, stats_ref):
    i = pl.program_id(0)

    @pl.when(i == 0)
    def _():
        stats_ref[...] = jnp.zeros((8, D), jnp.float32)

    dinv = 1.0 / jnp.maximum(deg_ref[...], 1.0)
    c0 = s0_ref[...] * dinv
    c1 = s1_ref[...] * dinv
    h0 = h2x_ref[0, :, :]
    h1 = h2x_ref[1, :, :]
    ssq = (jnp.sum(h0 * h0, axis=1, keepdims=True)
           + jnp.sum(h1 * h1, axis=1, keepdims=True)
           + jnp.sum(c0 * c0, axis=1, keepdims=True)
           + jnp.sum(c1 * c1, axis=1, keepdims=True))
    inv = 1.0 / jnp.maximum(jnp.sqrt(ssq), 1e-12)
    b_ref[:, 0 * H:1 * H] = h0 * inv
    b_ref[:, 1 * H:2 * H] = h1 * inv
    b_ref[:, 2 * H:3 * H] = c0 * inv
    b_ref[:, 3 * H:4 * H] = c1 * inv

    nrm = norm_ref[...]
    h2 = jnp.concatenate([c0 * nrm, c1 * nrm], axis=1)
    h2_ref[...] = h2
    stats_ref[0:1, :] += jnp.sum(h2, axis=0, keepdims=True)
    stats_ref[1:2, :] += jnp.sum(h2 * h2, axis=0, keepdims=True)


def _post(s0, s1, deg, h2x, norm):
    return pl.pallas_call(
        _post_body,
        grid=(NB,),
        in_specs=[
            pl.BlockSpec((RB, H), lambda i: (i, 0)),
            pl.BlockSpec((RB, H), lambda i: (i, 0)),
            pl.BlockSpec((RB, 1), lambda i: (i, 0)),
            pl.BlockSpec((2, RB, H), lambda i: (0, i, 0)),
            pl.BlockSpec((RB, 1), lambda i: (i, 0)),
        ],
        out_specs=[
            pl.BlockSpec((RB, 2 * D), lambda i: (i, 0)),
            pl.BlockSpec((RB, D), lambda i: (i, 0)),
            pl.BlockSpec((8, D), lambda i: (0, 0)),
        ],
        out_shape=[
            jax.ShapeDtypeStruct((N, 2 * D), jnp.float32),
            jax.ShapeDtypeStruct((N, D), jnp.float32),
            jax.ShapeDtypeStruct((8, D), jnp.float32),
        ],
    )(s0, s1, deg, h2x, norm)


def _bn_body(h2_ref, stats_ref, gamma_ref, beta_ref, h3_ref):
    mean = stats_ref[0:1, :] / float(N)
    var = stats_ref[1:2, :] / float(N) - mean * mean
    scale = gamma_ref[...] * lax.rsqrt(var + 1e-5)
    h3_ref[...] = (h2_ref[...] - mean) * scale + beta_ref[...]


def _bn(h2, stats, gamma, beta):
    return pl.pallas_call(
        _bn_body,
        grid=(NB,),
        in_specs=[
            pl.BlockSpec((RB, D), lambda i: (i, 0)),
            pl.BlockSpec((8, D), lambda i: (0, 0)),
            pl.BlockSpec((1, D), lambda i: (0, 0)),
            pl.BlockSpec((1, D), lambda i: (0, 0)),
        ],
        out_specs=pl.BlockSpec((RB, D), lambda i: (i, 0)),
        out_shape=jax.ShapeDtypeStruct((N, D), jnp.float32),
    )(h2, stats, gamma, beta)


def kernel(x, norm, gamma, beta, edge_index):
    edg = edge_index.astype(jnp.int32).reshape(2, NS, NST, IB, CH)
    h2x = _scale(x, norm)
    s0, s1, degq = _sc_agg(h2x[0], h2x[1], edg)
    deg = degq.reshape(HR * H)[:N].reshape(N, 1)
    b, h2, stats = _post(s0, s1, deg, h2x, norm)
    h3 = _bn(h2, stats, gamma.reshape(1, D), beta.reshape(1, D))
    return (h3, b)


# speedup vs baseline: 6.6415x; 1.0190x over previous
import jax
import jax.numpy as jnp
from jax import lax
from jax.experimental import pallas as pl
from jax.experimental.pallas import tpu as pltpu
from jax.experimental.pallas import tpu_sc as plsc

N = 10000
E = 160000
D = 256
H = 128
NS = 16
EPT = E // NS
CH = 80
NCH = EPT // CH
NST = 25
IB = NCH // NST
RPT = 632
RPT_L = N - (NS - 1) * RPT
HR = 80
L = 16
RB = 2000
NB = N // RB


def _scale_body(x_ref, norm_ref, h_ref):
    h = x_ref[...] * norm_ref[...]
    h_ref[0, :, :] = h[:, :H]
    h_ref[1, :, :] = h[:, H:]


def _scale(x, norm):
    return pl.pallas_call(
        _scale_body,
        grid=(NB,),
        in_specs=[
            pl.BlockSpec((RB, D), lambda i: (i, 0)),
            pl.BlockSpec((RB, 1), lambda i: (i, 0)),
        ],
        out_specs=pl.BlockSpec((2, RB, H), lambda i: (0, i, 0)),
        out_shape=jax.ShapeDtypeStruct((2, N, H), jnp.float32),
    )(x, norm)


def _sc_agg_body(h0, h1, edg, s0, s1, deg,
                 src_a, src_b, dst_a, dst_b, rows_a, rows_b, hist1, idx80,
                 acc_sh, sem, semi):
    c = lax.axis_index("c")
    tid = lax.axis_index("s")
    r0 = pl.multiple_of(tid * RPT, 8)
    nz = NS - 1

    zv = jnp.zeros((L,), jnp.float32)
    ov = jnp.ones((L,), jnp.float32)
    iv = lax.iota(jnp.int32, L)
    for i in range(8):
        for k in range(H // L):
            rows_a[i, k * L:(k + 1) * L] = zv
    for k in range(HR // L):
        idx80[k * L:(k + 1) * L] = iv + (k * L)

    def zh(j, carry):
        hist1[pl.ds(j * L, L)] = zv
        return carry
    lax.fori_loop(0, (HR * H) // L, zh, 0)

    def z8(j, carry):
        rj = pl.multiple_of(r0 + j * 8, 8)
        pltpu.sync_copy(rows_a.at[pl.ds(0, 8)], acc_sh.at[pl.ds(rj, 8)])
        return carry

    @pl.when(tid < nz)
    def _():
        lax.fori_loop(0, RPT // 8, z8, 0)

    @pl.when(tid == nz)
    def _():
        lax.fori_loop(0, RPT_L // 8, z8, 0)

    pltpu.async_copy(edg.at[0, tid, 0], src_a, semi)
    pltpu.async_copy(edg.at[1, tid, 0], dst_a, semi)

    plsc.subcore_barrier()

    def main_loop(h_half, count_deg):
        bufs = [rows_a, rows_b]

        def process_stage(src_v, dst_v, p, next_src, wait_fn):
            for j in range(IB):
                b = (p + j) % 2
                pltpu.make_async_copy(
                    h_half.at[src_v.at[j]], bufs[b], sem).wait()
                nxt = (p + j + 1) % 2
                if j + 1 < IB:
                    pltpu.async_copy(
                        h_half.at[src_v.at[j + 1]], bufs[nxt], sem)
                else:
                    if wait_fn is not None:
                        wait_fn()
                    if next_src is not None:
                        pltpu.async_copy(
                            h_half.at[next_src.at[0]], bufs[nxt], sem)
                pltpu.sync_copy(bufs[b], acc_sh.at[dst_v.at[j]], add=True)
                if count_deg:
                    for k in range(CH // L):
                        vec = dst_v[j, k * L:(k + 1) * L]
                        plsc.addupdate_scatter(hist1, [vec], ov)

        def wait_idx(g, sv, dv):
            pltpu.make_async_copy(edg.at[0, tid, g], sv, semi).wait()
            pltpu.make_async_copy(edg.at[1, tid, g], dv, semi).wait()

        def start_idx(g, sv, dv):
            pltpu.async_copy(edg.at[0, tid, g], sv, semi)
            pltpu.async_copy(edg.at[1, tid, g], dv, semi)

        wait_idx(0, src_a, dst_a)
        pltpu.async_copy(h_half.at[src_a.at[0]], bufs[0], sem)
        start_idx(1, src_b, dst_b)

        def stage_pair(t, carry):
            g = 2 * t
            process_stage(src_a, dst_a, 0, src_b,
                          lambda: wait_idx(g + 1, src_b, dst_b))
            start_idx(g + 2, src_a, dst_a)
            process_stage(src_b, dst_b, 1, src_a,
                          lambda: wait_idx(g + 2, src_a, dst_a))

            @pl.when(g + 3 < NST)
            def _():
                start_idx(g + 3, src_b, dst_b)
            return carry
        lax.fori_loop(0, (NST - 1) // 2, stage_pair, 0)
        process_stage(src_a, dst_a, 0, None, None)

    @pl.when(c == 0)
    def _():
        main_loop(h0, True)

    @pl.when(c == 1)
    def _():
        main_loop(h1, False)

    plsc.subcore_barrier()

    def write_out(cnt):
        @pl.when(c == 0)
        def _():
            pltpu.sync_copy(acc_sh.at[pl.ds(r0, cnt)], s0.at[pl.ds(r0, cnt)])

        @pl.when(c == 1)
        def _():
            pltpu.sync_copy(acc_sh.at[pl.ds(r0, cnt)], s1.at[pl.ds(r0, cnt)])

    @pl.when(tid < nz)
    def _():
        write_out(RPT)

    @pl.when(tid == nz)
    def _():
        write_out(RPT_L)

    @pl.when(c == 0)
    def _():
        @pl.when(tid == 0)
        def _():
            for i in range(8):
                for k in range(H // L):
                    rows_b[i, k * L:(k + 1) * L] = zv
            def zd(j, carry):
                rj = pl.multiple_of(j * 8, 8)
                pltpu.sync_copy(rows_b.at[pl.ds(0, 8)], acc_sh.at[pl.ds(rj, 8)])
                return carry
            lax.fori_loop(0, HR // 8, zd, 0)

        plsc.subcore_barrier()

        def cp(j, carry):
            for k in range(H // L):
                rows_a[j, k * L:(k + 1) * L] = hist1[pl.ds(j * H + k * L, L)]
            return carry
        lax.fori_loop(0, HR, cp, 0)
        pltpu.sync_copy(rows_a.at[pl.ds(0, HR)], acc_sh.at[idx80], add=True)

        plsc.subcore_barrier()

        @pl.when(tid == 0)
        def _():
            pltpu.sync_copy(acc_sh.at[pl.ds(0, HR)], deg)


def _sc_agg(h0, h1, edg):
    mesh = plsc.VectorSubcoreMesh(core_axis_name="c", subcore_axis_name="s",
                                  num_cores=2, num_subcores=NS)
    f = pl.kernel(
        _sc_agg_body,
        out_type=(
            jax.ShapeDtypeStruct((N, H), jnp.float32),
            jax.ShapeDtypeStruct((N, H), jnp.float32),
            jax.ShapeDtypeStruct((HR, H), jnp.float32),
        ),
        mesh=mesh,
        compiler_params=pltpu.CompilerParams(needs_layout_passes=False),
        scratch_types=[
            pltpu.VMEM((IB, CH), jnp.int32),
            pltpu.VMEM((IB, CH), jnp.int32),
            pltpu.VMEM((IB, CH), jnp.int32),
            pltpu.VMEM((IB, CH), jnp.int32),
            pltpu.VMEM((CH, H), jnp.float32),
            pltpu.VMEM((CH, H), jnp.float32),
            pltpu.VMEM((HR * H,), jnp.float32),
            pltpu.VMEM((HR,), jnp.int32),
            pltpu.VMEM_SHARED((N, H), jnp.float32),
            pltpu.SemaphoreType.DMA,
            pltpu.SemaphoreType.DMA,
        ],
    )
    return f(h0, h1, edg)


def _post_body(s0_ref, s1_ref, deg_ref, h2x_ref, norm_ref, b_ref, h2_ref, stats_ref):
    i = pl.program_id(0)

    @pl.when(i == 0)
    def _():
        stats_ref[...] = jnp.zeros((8, D), jnp.float32)

    dinv = 1.0 / jnp.maximum(deg_ref[...], 1.0)
    c0 = s0_ref[...] * dinv
    c1 = s1_ref[...] * dinv
    h0 = h2x_ref[0, :, :]
    h1 = h2x_ref[1, :, :]
    ssq = (jnp.sum(h0 * h0, axis=1, keepdims=True)
           + jnp.sum(h1 * h1, axis=1, keepdims=True)
           + jnp.sum(c0 * c0, axis=1, keepdims=True)
           + jnp.sum(c1 * c1, axis=1, keepdims=True))
    inv = 1.0 / jnp.maximum(jnp.sqrt(ssq), 1e-12)
    b_ref[:, 0 * H:1 * H] = h0 * inv
    b_ref[:, 1 * H:2 * H] = h1 * inv
    b_ref[:, 2 * H:3 * H] = c0 * inv
    b_ref[:, 3 * H:4 * H] = c1 * inv

    nrm = norm_ref[...]
    h2 = jnp.concatenate([c0 * nrm, c1 * nrm], axis=1)
    h2_ref[...] = h2
    stats_ref[0:1, :] += jnp.sum(h2, axis=0, keepdims=True)
    stats_ref[1:2, :] += jnp.sum(h2 * h2, axis=0, keepdims=True)


def _post(s0, s1, deg, h2x, norm):
    return pl.pallas_call(
        _post_body,
        grid=(NB,),
        in_specs=[
            pl.BlockSpec((RB, H), lambda i: (i, 0)),
            pl.BlockSpec((RB, H), lambda i: (i, 0)),
            pl.BlockSpec((RB, 1), lambda i: (i, 0)),
            pl.BlockSpec((2, RB, H), lambda i: (0, i, 0)),
            pl.BlockSpec((RB, 1), lambda i: (i, 0)),
        ],
        out_specs=[
            pl.BlockSpec((RB, 2 * D), lambda i: (i, 0)),
            pl.BlockSpec((RB, D), lambda i: (i, 0)),
            pl.BlockSpec((8, D), lambda i: (0, 0)),
        ],
        out_shape=[
            jax.ShapeDtypeStruct((N, 2 * D), jnp.float32),
            jax.ShapeDtypeStruct((N, D), jnp.float32),
            jax.ShapeDtypeStruct((8, D), jnp.float32),
        ],
    )(s0, s1, deg, h2x, norm)


def _bn_body(h2_ref, stats_ref, gamma_ref, beta_ref, h3_ref):
    mean = stats_ref[0:1, :] / float(N)
    var = stats_ref[1:2, :] / float(N) - mean * mean
    scale = gamma_ref[...] * lax.rsqrt(var + 1e-5)
    h3_ref[...] = (h2_ref[...] - mean) * scale + beta_ref[...]


def _bn(h2, stats, gamma, beta):
    return pl.pallas_call(
        _bn_body,
        grid=(NB,),
        in_specs=[
            pl.BlockSpec((RB, D), lambda i: (i, 0)),
            pl.BlockSpec((8, D), lambda i: (0, 0)),
            pl.BlockSpec((1, D), lambda i: (0, 0)),
            pl.BlockSpec((1, D), lambda i: (0, 0)),
        ],
        out_specs=pl.BlockSpec((RB, D), lambda i: (i, 0)),
        out_shape=jax.ShapeDtypeStruct((N, D), jnp.float32),
    )(h2, stats, gamma, beta)


def kernel(x, norm, gamma, beta, edge_index):
    edg = edge_index.astype(jnp.int32).reshape(2, NS, NST, IB, CH)
    h2x = _scale(x, norm)
    s0, s1, degq = _sc_agg(h2x[0], h2x[1], edg)
    deg = degq.reshape(HR * H)[:N].reshape(N, 1)
    b, h2, stats = _post(s0, s1, deg, h2x, norm)
    h3 = _bn(h2, stats, gamma.reshape(1, D), beta.reshape(1, D))
    return (h3, b)
